# baseline (speedup 1.0000x reference)
import functools

import jax
import jax.numpy as jnp
from jax import lax
from jax.experimental import pallas as pl
from jax.experimental.pallas import tpu as pltpu
from jax.experimental.pallas import tpu_sc as plsc

BATCH = 32
N_TS = 2048
N_EL = 256
N_OUT = 2048
NCOMB = 3

LANES = 16
K = 32
ROWS = K * NCOMB
VPR = N_EL // LANES

M_TC = 1024
N_SC = N_OUT - M_TC
CHUNKS = N_SC // K


def _tc_combine(x4d, wflat, idxflat):
  def body(idx_sref, w_sref, x_ref, o_ref):
    o = pl.program_id(0)
    c = pl.program_id(1)
    w = w_sref[o * NCOMB + c]
    blk = x_ref[:, 0, 0, :] * w

    @pl.when(c == 0)
    def _():
      o_ref[:, 0, 0, :] = blk

    @pl.when(c != 0)
    def _():
      o_ref[:, 0, 0, :] = o_ref[:, 0, 0, :] + blk

  grid_spec = pltpu.PrefetchScalarGridSpec(
      num_scalar_prefetch=2,
      grid=(M_TC, NCOMB),
      in_specs=[
          pl.BlockSpec((BATCH, 1, 1, N_EL),
                       lambda o, c, idx, w: (0, idx[o * NCOMB + c], 0, 0)),
      ],
      out_specs=pl.BlockSpec((BATCH, 1, 1, N_EL),
                             lambda o, c, idx, w: (0, o, 0, 0)),
  )
  out = pl.pallas_call(
      body,
      grid_spec=grid_spec,
      out_shape=jax.ShapeDtypeStruct((BATCH, M_TC, 1, N_EL), jnp.float32),
  )(idxflat, wflat, x4d)
  return out.reshape(BATCH, M_TC, N_EL)


def _sc_combine(x2d, wexp, idx2d):
  mesh = plsc.VectorSubcoreMesh(core_axis_name="c", subcore_axis_name="s")

  @functools.partial(
      pl.kernel,
      mesh=mesh,
      out_type=jax.ShapeDtypeStruct((BATCH, N_SC, N_EL), jnp.float32),
      scratch_types=[
          pltpu.VMEM((CHUNKS, ROWS), jnp.int32),
          pltpu.VMEM((2, ROWS, LANES), jnp.float32),
          pltpu.VMEM((2, ROWS, N_EL), jnp.float32),
          pltpu.VMEM((2, K, N_EL), jnp.float32),
          pltpu.SemaphoreType.DMA,
          pltpu.SemaphoreType.DMA,
          pltpu.SemaphoreType.DMA,
          pltpu.SemaphoreType.DMA,
          pltpu.SemaphoreType.DMA,
          pltpu.SemaphoreType.DMA,
      ],
  )
  def k(x_hbm, w_hbm, idx_hbm, out_hbm, idx_v, w_v, rows_v, out_v,
        semg0, semg1, semw0, semw1, semo0, semo1):
    semg = (semg0, semg1)
    semw = (semw0, semw1)
    semo = (semo0, semo1)
    cid = lax.axis_index("c")
    sid = lax.axis_index("s")
    b = sid * 2 + cid

    pltpu.sync_copy(idx_hbm, idx_v)

    off = jnp.full((LANES,), b * N_TS, jnp.int32)

    def add_off(t, carry):
      for j in range(ROWS // LANES):
        sl = pl.ds(j * LANES, LANES)
        idx_v[t, sl] = idx_v[t, sl] + off
      return carry

    lax.fori_loop(0, CHUNKS, add_off, 0)

    def issue(t, p):
      pltpu.async_copy(w_hbm.at[pl.ds(t * ROWS, ROWS)], w_v.at[p], semw[p])
      pltpu.async_copy(x_hbm.at[idx_v.at[t]], rows_v.at[p], semg[p])

    def out_slice(t):
      return out_hbm.at[b].at[pl.ds(t * K, K)]

    def combine(t, p):
      def one_out(o, carry):
        w0 = w_v[p, NCOMB * o, :]
        w1 = w_v[p, NCOMB * o + 1, :]
        w2 = w_v[p, NCOMB * o + 2, :]
        for v in range(VPR):
          sl = pl.ds(v * LANES, LANES)
          out_v[p, o, sl] = (rows_v[p, NCOMB * o, sl] * w0
                             + rows_v[p, NCOMB * o + 1, sl] * w1
                             + rows_v[p, NCOMB * o + 2, sl] * w2)
        return carry

      lax.fori_loop(0, K, one_out, 0)

    issue(0, 0)

    def gbody(g, carry):
      for p in range(2):
        t = 2 * g + p

        @pl.when(t + 1 < CHUNKS)
        def _():
          issue(t + 1, 1 - p)

        pltpu.make_async_copy(
            w_hbm.at[pl.ds(t * ROWS, ROWS)], w_v.at[p], semw[p]).wait()
        pltpu.make_async_copy(
            x_hbm.at[idx_v.at[t]], rows_v.at[p], semg[p]).wait()

        @pl.when(t >= 2)
        def _():
          pltpu.make_async_copy(out_v.at[p], out_slice(t - 2), semo[p]).wait()

        combine(t, p)
        pltpu.async_copy(out_v.at[p], out_slice(t), semo[p])
      return carry

    lax.fori_loop(0, CHUNKS // 2, gbody, 0)
    pltpu.make_async_copy(out_v.at[0], out_slice(CHUNKS - 2), semo[0]).wait()
    pltpu.make_async_copy(out_v.at[1], out_slice(CHUNKS - 1), semo[1]).wait()

  return k(x2d, wexp, idx2d)


def kernel(x, weights, selected_idx):
  wf32 = weights.astype(jnp.float32)
  idx_i32 = selected_idx.astype(jnp.int32)

  x4d = x.reshape(BATCH, N_TS, 1, N_EL)
  w_tc = wf32[:M_TC].reshape(-1)
  idx_tc = idx_i32[:M_TC].reshape(-1)
  out_tc = _tc_combine(x4d, w_tc, idx_tc)

  x2d = x.reshape(BATCH * N_TS, N_EL)
  wexp = jnp.broadcast_to(
      wf32[M_TC:].reshape(N_SC * NCOMB, 1), (N_SC * NCOMB, LANES))
  idx2d = idx_i32[M_TC:].reshape(CHUNKS, ROWS)
  out_sc = _sc_combine(x2d, wexp, idx2d)

  return jnp.concatenate([out_tc, out_sc], axis=1)


# speedup vs baseline: 5.9104x; 5.9104x over previous
import functools

import jax
import jax.numpy as jnp
from jax import lax
from jax.experimental import pallas as pl
from jax.experimental.pallas import tpu as pltpu
from jax.experimental.pallas import tpu_sc as plsc

BATCH = 32
N_TS = 2048
N_EL = 256
N_OUT = 2048
NCOMB = 3

LANES = 16
K = 32
CHUNKS = N_OUT // K
ROWS = K * NCOMB
VPR = N_EL // LANES


def _sc_combine(x2d, wexp, idx2d):
  mesh = plsc.VectorSubcoreMesh(core_axis_name="c", subcore_axis_name="s")

  @functools.partial(
      pl.kernel,
      mesh=mesh,
      out_type=jax.ShapeDtypeStruct((BATCH, N_OUT, N_EL), jnp.float32),
      scratch_types=[
          pltpu.VMEM((CHUNKS, ROWS), jnp.int32),
          pltpu.VMEM((2, ROWS, LANES), jnp.float32),
          pltpu.VMEM((2, ROWS, N_EL), jnp.float32),
          pltpu.VMEM((2, K, N_EL), jnp.float32),
          pltpu.SemaphoreType.DMA,
          pltpu.SemaphoreType.DMA,
          pltpu.SemaphoreType.DMA,
          pltpu.SemaphoreType.DMA,
          pltpu.SemaphoreType.DMA,
          pltpu.SemaphoreType.DMA,
      ],
  )
  def k(x_hbm, w_hbm, idx_hbm, out_hbm, idx_v, w_v, rows_v, out_v,
        semg0, semg1, semw0, semw1, semo0, semo1):
    semg = (semg0, semg1)
    semw = (semw0, semw1)
    semo = (semo0, semo1)
    cid = lax.axis_index("c")
    sid = lax.axis_index("s")
    b = sid * 2 + cid

    pltpu.sync_copy(idx_hbm, idx_v)

    off = jnp.full((LANES,), b * N_TS, jnp.int32)

    def add_off(t, carry):
      for j in range(ROWS // LANES):
        sl = pl.ds(j * LANES, LANES)
        idx_v[t, sl] = idx_v[t, sl] + off
      return carry

    lax.fori_loop(0, CHUNKS, add_off, 0)

    def issue(t, p):
      pltpu.async_copy(w_hbm.at[pl.ds(t * ROWS, ROWS)], w_v.at[p], semw[p])
      pltpu.async_copy(x_hbm.at[idx_v.at[t]], rows_v.at[p], semg[p])

    def out_slice(t):
      return out_hbm.at[b].at[pl.ds(t * K, K)]

    def combine(t, p):
      def one_out(o, carry):
        w0 = w_v[p, NCOMB * o, :]
        w1 = w_v[p, NCOMB * o + 1, :]
        w2 = w_v[p, NCOMB * o + 2, :]
        for v in range(VPR):
          sl = pl.ds(v * LANES, LANES)
          out_v[p, o, sl] = (rows_v[p, NCOMB * o, sl] * w0
                             + rows_v[p, NCOMB * o + 1, sl] * w1
                             + rows_v[p, NCOMB * o + 2, sl] * w2)
        return carry

      lax.fori_loop(0, K, one_out, 0)

    issue(0, 0)

    def gbody(g, carry):
      for p in range(2):
        t = 2 * g + p

        @pl.when(t + 1 < CHUNKS)
        def _():
          issue(t + 1, 1 - p)

        pltpu.make_async_copy(
            w_hbm.at[pl.ds(t * ROWS, ROWS)], w_v.at[p], semw[p]).wait()
        pltpu.make_async_copy(
            x_hbm.at[idx_v.at[t]], rows_v.at[p], semg[p]).wait()

        @pl.when(t >= 2)
        def _():
          pltpu.make_async_copy(out_v.at[p], out_slice(t - 2), semo[p]).wait()

        combine(t, p)
        pltpu.async_copy(out_v.at[p], out_slice(t), semo[p])
      return carry

    lax.fori_loop(0, CHUNKS // 2, gbody, 0)
    pltpu.make_async_copy(out_v.at[0], out_slice(CHUNKS - 2), semo[0]).wait()
    pltpu.make_async_copy(out_v.at[1], out_slice(CHUNKS - 1), semo[1]).wait()

  return k(x2d, wexp, idx2d)


def kernel(x, weights, selected_idx):
  x2d = x.reshape(BATCH * N_TS, N_EL)
  wexp = jnp.broadcast_to(
      weights.astype(jnp.float32).reshape(N_OUT * NCOMB, 1), (N_OUT * NCOMB, LANES))
  idx2d = selected_idx.astype(jnp.int32).reshape(CHUNKS, ROWS)
  return _sc_combine(x2d, wexp, idx2d)


# speedup vs baseline: 8.8713x; 1.5010x over previous
import functools

import jax
import jax.numpy as jnp
from jax import lax
from jax.experimental import pallas as pl
from jax.experimental.pallas import tpu as pltpu
from jax.experimental.pallas import tpu_sc as plsc

BATCH = 32
N_TS = 2048
N_EL = 256
N_OUT = 2048
NCOMB = 3

LANES = 16
K = 32
CHUNKS = N_OUT // K
ROWS = K * NCOMB
VPR = N_EL // LANES


def _sc_combine(x2d, wexp, idx2d):
  mesh = plsc.VectorSubcoreMesh(core_axis_name="c", subcore_axis_name="s")

  @functools.partial(
      pl.kernel,
      mesh=mesh,
      out_type=jax.ShapeDtypeStruct((BATCH, N_OUT, N_EL), jnp.float32),
      scratch_types=[
          pltpu.VMEM((CHUNKS, ROWS), jnp.int32),
          pltpu.VMEM((2, ROWS, LANES), jnp.float32),
          pltpu.VMEM((2, ROWS, N_EL), jnp.float32),
          pltpu.VMEM((2, K, N_EL), jnp.float32),
          pltpu.SemaphoreType.DMA,
          pltpu.SemaphoreType.DMA,
          pltpu.SemaphoreType.DMA,
          pltpu.SemaphoreType.DMA,
          pltpu.SemaphoreType.DMA,
          pltpu.SemaphoreType.DMA,
      ],
  )
  def k(x_hbm, w_hbm, idx_hbm, out_hbm, idx_v, w_v, rows_v, out_v,
        semg0, semg1, semw0, semw1, semo0, semo1):
    semg = (semg0, semg1)
    semw = (semw0, semw1)
    semo = (semo0, semo1)
    cid = lax.axis_index("c")
    sid = lax.axis_index("s")
    b = sid * 2 + cid

    pltpu.sync_copy(idx_hbm, idx_v)

    off = jnp.full((LANES,), b * N_TS, jnp.int32)

    def add_off(t, carry):
      for j in range(ROWS // LANES):
        sl = pl.ds(j * LANES, LANES)
        idx_v[t, sl] = idx_v[t, sl] + off
      return carry

    lax.fori_loop(0, CHUNKS, add_off, 0)

    def issue(t, p):
      pltpu.async_copy(w_hbm.at[pl.ds(t * ROWS, ROWS)], w_v.at[p], semw[p])
      pltpu.async_copy(x_hbm.at[idx_v.at[t]], rows_v.at[p], semg[p])

    def out_slice(t):
      return out_hbm.at[b].at[pl.ds(t * K, K)]

    def combine(t, p):
      @plsc.parallel_loop(0, K, unroll=2)
      def one_out(o):
        w0 = w_v[p, NCOMB * o, :]
        w1 = w_v[p, NCOMB * o + 1, :]
        w2 = w_v[p, NCOMB * o + 2, :]
        for v in range(VPR):
          sl = pl.ds(v * LANES, LANES)
          out_v[p, o, sl] = (rows_v[p, NCOMB * o, sl] * w0
                             + rows_v[p, NCOMB * o + 1, sl] * w1
                             + rows_v[p, NCOMB * o + 2, sl] * w2)

    issue(0, 0)

    def gbody(g, carry):
      for p in range(2):
        t = 2 * g + p

        @pl.when(t + 1 < CHUNKS)
        def _():
          issue(t + 1, 1 - p)

        pltpu.make_async_copy(
            w_hbm.at[pl.ds(t * ROWS, ROWS)], w_v.at[p], semw[p]).wait()
        pltpu.make_async_copy(
            x_hbm.at[idx_v.at[t]], rows_v.at[p], semg[p]).wait()

        @pl.when(t >= 2)
        def _():
          pltpu.make_async_copy(out_v.at[p], out_slice(t - 2), semo[p]).wait()

        combine(t, p)
        pltpu.async_copy(out_v.at[p], out_slice(t), semo[p])
      return carry

    lax.fori_loop(0, CHUNKS // 2, gbody, 0)
    pltpu.make_async_copy(out_v.at[0], out_slice(CHUNKS - 2), semo[0]).wait()
    pltpu.make_async_copy(out_v.at[1], out_slice(CHUNKS - 1), semo[1]).wait()

  return k(x2d, wexp, idx2d)


def kernel(x, weights, selected_idx):
  x2d = x.reshape(BATCH * N_TS, N_EL)
  wexp = jnp.broadcast_to(
      weights.astype(jnp.float32).reshape(N_OUT * NCOMB, 1), (N_OUT * NCOMB, LANES))
  idx2d = selected_idx.astype(jnp.int32).reshape(CHUNKS, ROWS)
  return _sc_combine(x2d, wexp, idx2d)
